# baseline (speedup 1.0000x reference)
import functools

import jax
import jax.numpy as jnp
from jax import lax
from jax.experimental import pallas as pl
from jax.experimental.pallas import tpu as pltpu
from jax.experimental.pallas import tpu_sc as plsc

WORD_DIM = 64
DIST_DIM = 32
OUT_DIM = WORD_DIM + 2 * DIST_DIM
NVOC = 1000
BLK = 400


@functools.lru_cache(maxsize=None)
def _make_sc_kernel(N: int):
    info = plsc.get_sparse_core_info()
    NC, NS = info.num_cores, info.num_subcores
    NW = NC * NS
    assert N % (NW * 2 * BLK) == 0
    chunk = N // NW
    nblk = chunk // BLK
    mesh = plsc.VectorSubcoreMesh(core_axis_name="c", subcore_axis_name="s")

    buf_types = []
    for _ in range(2):
        buf_types += [
            pltpu.VMEM((BLK, WORD_DIM), jnp.float32),
            pltpu.VMEM((BLK, DIST_DIM), jnp.float32),
            pltpu.VMEM((BLK, DIST_DIM), jnp.float32),
            pltpu.VMEM((BLK,), jnp.int32),
            pltpu.VMEM((BLK,), jnp.int32),
            pltpu.SemaphoreType.DMA,
            pltpu.SemaphoreType.DMA,
            pltpu.SemaphoreType.DMA,
        ]

    @functools.partial(
        pl.kernel,
        mesh=mesh,
        out_type=jax.ShapeDtypeStruct((N, OUT_DIM), jnp.float32),
        compiler_params=pltpu.CompilerParams(use_tc_tiling_on_sc=False),
        scratch_types=[
            pltpu.VMEM((chunk,), jnp.int32),
            pltpu.VMEM_SHARED((2 * NVOC, DIST_DIM), jnp.float32),
        ] + buf_types,
    )
    def k(widx_hbm, eidx_hbm, nidx_hbm, wtab, etab, ntab, out_hbm,
          widx_v, entnum_sh, *bufs):
        sets = [bufs[8 * d:8 * d + 8] for d in range(2)]
        cid = lax.axis_index("c")
        sid = lax.axis_index("s")
        wid = sid * NC + cid
        base = wid * chunk

        @pl.when(sid == 0)
        def _():
            pltpu.sync_copy(etab, entnum_sh.at[pl.ds(0, NVOC)])
            pltpu.sync_copy(ntab, entnum_sh.at[pl.ds(NVOC, NVOC)])

        pltpu.sync_copy(widx_hbm.at[pl.ds(base, chunk)], widx_v)
        plsc.subcore_barrier()

        def fire_hbm(b, s):
            wbuf, _, _, eidx, nidx, gsem, _, _ = s
            off = b * BLK
            pltpu.async_copy(wtab.at[widx_v.at[pl.ds(off, BLK)]], wbuf, gsem)
            pltpu.async_copy(eidx_hbm.at[pl.ds(base + off, BLK)], eidx, gsem)
            pltpu.async_copy(nidx_hbm.at[pl.ds(base + off, BLK)], nidx, gsem)

        def wait_hbm(b, s):
            wbuf, _, _, eidx, nidx, gsem, _, _ = s
            off = b * BLK
            pltpu.make_async_copy(wtab.at[widx_v.at[pl.ds(off, BLK)]], wbuf, gsem).wait()
            pltpu.make_async_copy(eidx_hbm.at[pl.ds(base + off, BLK)], eidx, gsem).wait()
            pltpu.make_async_copy(nidx_hbm.at[pl.ds(base + off, BLK)], nidx, gsem).wait()

        def process(b, s):
            wbuf, ebuf, nbuf, eidx, nidx, _, ssem, wsem = s
            wait_hbm(b, s)
            ce = pltpu.async_copy(entnum_sh.at[eidx], ebuf, ssem)
            cn = pltpu.async_copy(entnum_sh.at[nidx], nbuf, ssem)
            ce.wait()
            cn.wait()
            row = base + b * BLK
            pltpu.async_copy(
                wbuf, out_hbm.at[pl.ds(row, BLK), pl.ds(0, WORD_DIM)], wsem)
            pltpu.async_copy(
                ebuf, out_hbm.at[pl.ds(row, BLK), pl.ds(WORD_DIM, DIST_DIM)], wsem)
            pltpu.async_copy(
                nbuf, out_hbm.at[pl.ds(row, BLK), pl.ds(WORD_DIM + DIST_DIM, DIST_DIM)], wsem)

        def drain_writes(b, s):
            wbuf, ebuf, nbuf, _, _, _, _, wsem = s
            row = base + b * BLK
            pltpu.make_async_copy(
                wbuf, out_hbm.at[pl.ds(row, BLK), pl.ds(0, WORD_DIM)], wsem).wait()
            pltpu.make_async_copy(
                ebuf, out_hbm.at[pl.ds(row, BLK), pl.ds(WORD_DIM, DIST_DIM)], wsem).wait()
            pltpu.make_async_copy(
                nbuf, out_hbm.at[pl.ds(row, BLK), pl.ds(WORD_DIM + DIST_DIM, DIST_DIM)], wsem).wait()

        for b in range(nblk + 1):
            if b < nblk:
                if b >= 2:
                    drain_writes(b - 2, sets[b % 2])
                fire_hbm(b, sets[b % 2])
            if b >= 1:
                process(b - 1, sets[(b - 1) % 2])
        drain_writes(nblk - 2, sets[nblk % 2])
        drain_writes(nblk - 1, sets[(nblk - 1) % 2])

    return k


def kernel(sents, entdists, numdists, word_table, ent_table, num_table):
    B, L = sents.shape
    N = B * L
    widx = sents.reshape(N).astype(jnp.int32)
    eidx = entdists.reshape(N).astype(jnp.int32)
    nidx = numdists.reshape(N).astype(jnp.int32) + NVOC
    out = _make_sc_kernel(N)(widx, eidx, nidx, word_table, ent_table, num_table)
    return out.reshape(B, L, OUT_DIM)
